# baseline (speedup 1.0000x reference)
import functools

import jax
import jax.numpy as jnp
from jax import lax
from jax.experimental import pallas as pl
from jax.experimental.pallas import tpu as pltpu
from jax.experimental.pallas import tpu_sc as plsc

N = 10000
NP = 10240
D = 128
DH = D // 2
E = 320000
NC, NS, L = 2, 16, 16
CH = 128
CPT = 157
E2 = NS * CPT * CH
EPAD = E2 - E
RPT = NP // NS
ZR = 128
BN = 1024
NB = 4

_mesh = plsc.VectorSubcoreMesh(core_axis_name="c", subcore_axis_name="s")
_mesh1 = plsc.VectorSubcoreMesh(core_axis_name="c", subcore_axis_name="s",
                                num_cores=1)


@functools.partial(
    pl.kernel,
    out_type=(
        jax.ShapeDtypeStruct((NC, NP, DH), jnp.float32),
        jax.ShapeDtypeStruct((NP,), jnp.float32),
        jax.ShapeDtypeStruct((NP,), jnp.float32),
    ),
    mesh=_mesh,
    scratch_types=[
        pltpu.VMEM((CPT, CH), jnp.int32),
        pltpu.VMEM((CPT, CH), jnp.int32),
    ] + [pltpu.VMEM((CH, DH), jnp.float32)] * NB + [
        pltpu.VMEM((CH,), jnp.float32),
        pltpu.VMEM((ZR, DH), jnp.float32),
        pltpu.VMEM((RPT,), jnp.float32),
        pltpu.VMEM_SHARED((NP, DH), jnp.float32),
        pltpu.VMEM_SHARED((NP,), jnp.float32),
    ] + [pltpu.SemaphoreType.DMA] * NB,
    compiler_params=pltpu.CompilerParams(use_tc_tiling_on_sc=False),
)
def _sc_agg_rows(xr_hbm, srcA_hbm, srcB_hbm, dst_hbm, acc_out, deg0_out,
                 deg1_out, src_buf, dst_buf, rows_0, rows_1, rows_2, rows_3,
                 ones_v, zrow, zdeg, acc_sh, deg_sh,
                 sem_0, sem_1, sem_2, sem_3):
    cid = lax.axis_index("c")
    sid = lax.axis_index("s")
    bufs = [rows_0, rows_1, rows_2, rows_3]
    sems = [sem_0, sem_1, sem_2, sem_3]

    def zfill(r, _):
        for k in range(DH // L):
            zrow[r, pl.ds(k * L, L)] = jnp.zeros((L,), jnp.float32)
        return 0
    lax.fori_loop(0, ZR, zfill, 0)
    for k in range(RPT // L):
        zdeg[pl.ds(k * L, L)] = jnp.zeros((L,), jnp.float32)
    for k in range(CH // L):
        ones_v[pl.ds(k * L, L)] = jnp.ones((L,), jnp.float32)

    for k in range(RPT // ZR):
        pltpu.sync_copy(zrow, acc_sh.at[pl.ds(sid * RPT + k * ZR, ZR)])
    pltpu.sync_copy(zdeg, deg_sh.at[pl.ds(sid * RPT, RPT)])
    plsc.subcore_barrier()

    @pl.when(cid == 0)
    def _():
        pltpu.sync_copy(srcA_hbm.at[sid], src_buf)

    @pl.when(cid == 1)
    def _():
        pltpu.sync_copy(srcB_hbm.at[sid], src_buf)
    pltpu.sync_copy(dst_hbm.at[sid], dst_buf)

    def start(j, b):
        pltpu.async_copy(xr_hbm.at[src_buf.at[j]], bufs[b], sems[b])

    def finish(j, b):
        pltpu.make_async_copy(xr_hbm.at[src_buf.at[j]], bufs[b],
                              sems[b]).wait()

    def consume(j, b, k):
        pltpu.sync_copy(bufs[b], acc_sh.at[dst_buf.at[j]], add=True)

        @pl.when(cid == (k % 2))
        def _():
            pltpu.sync_copy(ones_v, deg_sh.at[dst_buf.at[j]], add=True)

    for k in range(NB - 1):
        start(k, k)

    def body(i, _):
        for k in range(NB):
            j = NB * i + k

            @pl.when(j + NB - 1 < CPT)
            def _():
                start(j + NB - 1, (k + NB - 1) % NB)
            finish(j, k)
            consume(j, k, k)
        return 0
    lax.fori_loop(0, CPT // NB, body, 0)
    tail = (CPT // NB) * NB
    for j in range(tail, CPT):
        finish(j, j % NB)
        consume(j, j % NB, j)
    plsc.subcore_barrier()

    pltpu.sync_copy(acc_sh.at[pl.ds(sid * RPT, RPT)],
                    acc_out.at[cid, pl.ds(sid * RPT, RPT)])

    @pl.when(cid == 0)
    def _():
        pltpu.sync_copy(deg_sh.at[pl.ds(sid * RPT, RPT)],
                        deg0_out.at[pl.ds(sid * RPT, RPT)])

    @pl.when(cid == 1)
    def _():
        pltpu.sync_copy(deg_sh.at[pl.ds(sid * RPT, RPT)],
                        deg1_out.at[pl.ds(sid * RPT, RPT)])


@functools.partial(
    pl.kernel,
    out_type=jax.ShapeDtypeStruct((NP,), jnp.float32),
    mesh=_mesh1,
    scratch_types=[
        pltpu.VMEM((CPT, CH), jnp.int32),
        pltpu.VMEM((CPT, CH), jnp.int32),
        pltpu.VMEM((2 * NP,), jnp.float32),
        pltpu.VMEM((CPT, CH), jnp.float32),
        pltpu.VMEM((RPT,), jnp.float32),
        pltpu.VMEM((RPT,), jnp.float32),
        pltpu.VMEM((RPT,), jnp.float32),
        pltpu.VMEM((RPT,), jnp.float32),
        pltpu.VMEM((RPT,), jnp.float32),
        pltpu.VMEM_SHARED((NP,), jnp.float32),
    ] + [pltpu.SemaphoreType.DMA] * NB,
    compiler_params=pltpu.CompilerParams(use_tc_tiling_on_sc=False,
                                         needs_layout_passes=False),
)
def _sc_agg_scalar(st_hbm, degc_hbm, s2rb_hbm, src_hbm, dst_hbm, out_hbm,
                   src_buf, dst_buf, s_tile, vals_all, zdeg,
                   agg_t, deg_t, s2r_t, out_t, agg_sh,
                   sem_0, sem_1, sem_2, sem_3):
    sid = lax.axis_index("s")
    sems = [sem_0, sem_1, sem_2, sem_3]

    for k in range(RPT // L):
        zdeg[pl.ds(k * L, L)] = jnp.zeros((L,), jnp.float32)
    pltpu.sync_copy(zdeg, agg_sh.at[pl.ds(sid * RPT, RPT)])
    plsc.subcore_barrier()

    pltpu.sync_copy(st_hbm, s_tile)
    pltpu.sync_copy(src_hbm.at[sid], src_buf)
    pltpu.sync_copy(dst_hbm.at[sid], dst_buf)

    def sstart(j, b):
        pltpu.async_copy(vals_all.at[j], agg_sh.at[dst_buf.at[j]], sems[b],
                         add=True)

    def sfinish(j, b):
        pltpu.make_async_copy(vals_all.at[j], agg_sh.at[dst_buf.at[j]],
                              sems[b]).wait()

    def compute(j):
        for g in range(CH // L):
            idx = src_buf[j, pl.ds(g * L, L)]
            vals_all[j, pl.ds(g * L, L)] = plsc.load_gather(s_tile, [idx])

    def sbody(i, _):
        for k in range(NB):
            j = NB * i + k
            compute(j)

            @pl.when(j >= NB)
            def _():
                sfinish(j - NB, k)
            sstart(j, k)
        return 0
    lax.fori_loop(0, CPT // NB, sbody, 0)
    tail = (CPT // NB) * NB
    for j in range(tail, CPT):
        compute(j)
        sfinish(j - NB, j % NB)
        sstart(j, j % NB)
    for j in range(CPT - NB, CPT):
        sfinish(j, j % NB)
    plsc.subcore_barrier()

    pltpu.sync_copy(agg_sh.at[pl.ds(sid * RPT, RPT)], agg_t)
    pltpu.sync_copy(degc_hbm.at[pl.ds(sid * RPT, RPT)], deg_t)
    pltpu.sync_copy(s2rb_hbm.at[pl.ds(sid * RPT, RPT)], s2r_t)

    def ebody(k, _):
        a = agg_t[pl.ds(k * L, L)]
        d = deg_t[pl.ds(k * L, L)]
        out_t[pl.ds(k * L, L)] = a / d + s2r_t[pl.ds(k * L, L)]
        return 0
    lax.fori_loop(0, RPT // L, ebody, 0)
    pltpu.sync_copy(out_t, out_hbm.at[pl.ds(sid * RPT, RPT)])


def _tc_dense_body(acc_ref, deg0_ref, deg1_ref, x_ref, w1lt_ref, w1rt_ref,
                   b1l_ref, w2lt_ref, w2rt_ref, b2l_ref,
                   s2l_ref, s2r_ref, degc_ref):
    d = jnp.maximum(deg0_ref[...] + deg1_ref[...], 1.0)
    degc_ref[...] = d
    m0 = acc_ref[0] / d[:, None]
    m1 = acc_ref[1] / d[:, None]
    w1lt = w1lt_ref[...]
    h = (jnp.dot(m0, w1lt[:DH], preferred_element_type=jnp.float32)
         + jnp.dot(m1, w1lt[DH:], preferred_element_type=jnp.float32)
         + jnp.dot(x_ref[...], w1rt_ref[...], preferred_element_type=jnp.float32)
         + b1l_ref[...])
    h = jnp.maximum(h, 0.0)
    s2l_ref[...] = jnp.dot(h, w2lt_ref[...],
                           preferred_element_type=jnp.float32)[:, 0]
    s2r_ref[...] = (jnp.dot(h, w2rt_ref[...],
                            preferred_element_type=jnp.float32)[:, 0]
                    + b2l_ref[0, 0])


_tc_dense = pl.pallas_call(
    _tc_dense_body,
    grid=(NP // BN,),
    in_specs=[
        pl.BlockSpec((NC, BN, DH), lambda i: (0, i, 0)),
        pl.BlockSpec((BN,), lambda i: (i,)),
        pl.BlockSpec((BN,), lambda i: (i,)),
        pl.BlockSpec((BN, D), lambda i: (i, 0)),
        pl.BlockSpec((D, D), lambda i: (0, 0)),
        pl.BlockSpec((D, D), lambda i: (0, 0)),
        pl.BlockSpec((1, D), lambda i: (0, 0)),
        pl.BlockSpec((D, 1), lambda i: (0, 0)),
        pl.BlockSpec((D, 1), lambda i: (0, 0)),
        pl.BlockSpec((1, 1), lambda i: (0, 0)),
    ],
    out_specs=[
        pl.BlockSpec((BN,), lambda i: (i,)),
        pl.BlockSpec((BN,), lambda i: (i,)),
        pl.BlockSpec((BN,), lambda i: (i,)),
    ],
    out_shape=[
        jax.ShapeDtypeStruct((NP,), jnp.float32),
        jax.ShapeDtypeStruct((NP,), jnp.float32),
        jax.ShapeDtypeStruct((NP,), jnp.float32),
    ],
)


def kernel(x, edge_index, W1l, b1l, W1r, W2l, b2l, W2r):
    srcp = jnp.concatenate([edge_index[0],
                            jnp.zeros((EPAD,), jnp.int32)])
    dstp = jnp.concatenate([edge_index[1],
                            jnp.full((EPAD,), N, jnp.int32)])
    srcA = (srcp * 2).reshape(NS, CPT, CH)
    srcB = (srcp * 2 + 1).reshape(NS, CPT, CH)
    dst4 = dstp.reshape(NS, CPT, CH)

    xr = x.reshape(2 * N, DH)
    acc, deg0, deg1 = _sc_agg_rows(xr, srcA, srcB, dst4)

    s2l, s2rb, degc = _tc_dense(acc, deg0, deg1, x,
                                W1l.T, W1r.T, b1l.reshape(1, D),
                                W2l.T, W2r.T, b2l.reshape(1, 1))

    st = jnp.stack([s2l, s2l], axis=1).reshape(2 * NP)
    out = _sc_agg_scalar(st, degc, s2rb, srcA, dst4)
    return out[:N]


# speedup vs baseline: 1.0064x; 1.0064x over previous
import functools

import jax
import jax.numpy as jnp
from jax import lax
from jax.experimental import pallas as pl
from jax.experimental.pallas import tpu as pltpu
from jax.experimental.pallas import tpu_sc as plsc

N = 10000
NP = 10240
D = 128
DH = D // 2
E = 320000
NC, NS, L = 2, 16, 16
CH = 128
CPT = 157
E2 = NS * CPT * CH
EPAD = E2 - E
RPT = NP // NS
ZR = 128
BN = 1024
NB = 4

_mesh = plsc.VectorSubcoreMesh(core_axis_name="c", subcore_axis_name="s")
_mesh1 = plsc.VectorSubcoreMesh(core_axis_name="c", subcore_axis_name="s",
                                num_cores=1)


@functools.partial(
    pl.kernel,
    out_type=(
        jax.ShapeDtypeStruct((NC, NP, DH), jnp.float32),
        jax.ShapeDtypeStruct((NP,), jnp.float32),
        jax.ShapeDtypeStruct((NP,), jnp.float32),
    ),
    mesh=_mesh,
    scratch_types=[
        pltpu.VMEM((CPT, CH), jnp.int32),
        pltpu.VMEM((CPT, CH), jnp.int32),
    ] + [pltpu.VMEM((CH, DH), jnp.float32)] * NB + [
        pltpu.VMEM((CH,), jnp.float32),
        pltpu.VMEM((ZR, DH), jnp.float32),
        pltpu.VMEM((RPT,), jnp.float32),
        pltpu.VMEM_SHARED((NP, DH), jnp.float32),
        pltpu.VMEM_SHARED((NP,), jnp.float32),
    ] + [pltpu.SemaphoreType.DMA] * NB,
    compiler_params=pltpu.CompilerParams(use_tc_tiling_on_sc=False),
)
def _sc_agg_rows(xr_hbm, srcA_hbm, srcB_hbm, dst_hbm, acc_out, deg0_out,
                 deg1_out, src_buf, dst_buf, rows_0, rows_1, rows_2, rows_3,
                 ones_v, zrow, zdeg, acc_sh, deg_sh,
                 sem_0, sem_1, sem_2, sem_3):
    cid = lax.axis_index("c")
    sid = lax.axis_index("s")
    bufs = [rows_0, rows_1, rows_2, rows_3]
    sems = [sem_0, sem_1, sem_2, sem_3]

    def zfill(r, _):
        for k in range(DH // L):
            zrow[r, pl.ds(k * L, L)] = jnp.zeros((L,), jnp.float32)
        return 0
    lax.fori_loop(0, ZR, zfill, 0)
    for k in range(RPT // L):
        zdeg[pl.ds(k * L, L)] = jnp.zeros((L,), jnp.float32)
    for k in range(CH // L):
        ones_v[pl.ds(k * L, L)] = jnp.ones((L,), jnp.float32)

    for k in range(RPT // ZR):
        pltpu.sync_copy(zrow, acc_sh.at[pl.ds(sid * RPT + k * ZR, ZR)])
    pltpu.sync_copy(zdeg, deg_sh.at[pl.ds(sid * RPT, RPT)])
    plsc.subcore_barrier()

    @pl.when(cid == 0)
    def _():
        pltpu.sync_copy(srcA_hbm.at[sid], src_buf)

    @pl.when(cid == 1)
    def _():
        pltpu.sync_copy(srcB_hbm.at[sid], src_buf)
    pltpu.sync_copy(dst_hbm.at[sid], dst_buf)

    def start(j, b):
        pltpu.async_copy(xr_hbm.at[src_buf.at[j]], bufs[b], sems[b])

    def finish(j, b):
        pltpu.make_async_copy(xr_hbm.at[src_buf.at[j]], bufs[b],
                              sems[b]).wait()

    def consume(j, b, k):
        pltpu.sync_copy(bufs[b], acc_sh.at[dst_buf.at[j]], add=True)

        @pl.when(cid == (k % 2))
        def _():
            pltpu.sync_copy(ones_v, deg_sh.at[dst_buf.at[j]], add=True)

    for k in range(NB - 1):
        start(k, k)

    def body(i, _):
        for k in range(NB):
            j = NB * i + k

            @pl.when(j + NB - 1 < CPT)
            def _():
                start(j + NB - 1, (k + NB - 1) % NB)
            finish(j, k)
            consume(j, k, k)
        return 0
    lax.fori_loop(0, CPT // NB, body, 0)
    tail = (CPT // NB) * NB
    for j in range(tail, CPT):
        finish(j, j % NB)
        consume(j, j % NB, j)
    plsc.subcore_barrier()

    pltpu.sync_copy(acc_sh.at[pl.ds(sid * RPT, RPT)],
                    acc_out.at[cid, pl.ds(sid * RPT, RPT)])

    @pl.when(cid == 0)
    def _():
        pltpu.sync_copy(deg_sh.at[pl.ds(sid * RPT, RPT)],
                        deg0_out.at[pl.ds(sid * RPT, RPT)])

    @pl.when(cid == 1)
    def _():
        pltpu.sync_copy(deg_sh.at[pl.ds(sid * RPT, RPT)],
                        deg1_out.at[pl.ds(sid * RPT, RPT)])


@functools.partial(
    pl.kernel,
    out_type=jax.ShapeDtypeStruct((NP,), jnp.float32),
    mesh=_mesh1,
    scratch_types=[
        pltpu.VMEM((CPT, CH), jnp.int32),
        pltpu.VMEM((CPT, CH), jnp.int32),
        pltpu.VMEM((2 * NP,), jnp.float32),
        pltpu.VMEM((CPT, CH), jnp.float32),
        pltpu.VMEM((RPT,), jnp.float32),
        pltpu.VMEM((RPT,), jnp.float32),
        pltpu.VMEM((RPT,), jnp.float32),
        pltpu.VMEM((RPT,), jnp.float32),
        pltpu.VMEM((RPT,), jnp.float32),
        pltpu.VMEM_SHARED((NP,), jnp.float32),
    ] + [pltpu.SemaphoreType.DMA] * NB,
    compiler_params=pltpu.CompilerParams(use_tc_tiling_on_sc=False,
                                         needs_layout_passes=False),
)
def _sc_agg_scalar(st_hbm, degc_hbm, s2rb_hbm, src_hbm, dst_hbm, out_hbm,
                   src_buf, dst_buf, s_tile, vals_all, zdeg,
                   agg_t, deg_t, s2r_t, out_t, agg_sh,
                   sem_0, sem_1, sem_2, sem_3):
    sid = lax.axis_index("s")
    sems = [sem_0, sem_1, sem_2, sem_3]

    for k in range(RPT // L):
        zdeg[pl.ds(k * L, L)] = jnp.zeros((L,), jnp.float32)
    pltpu.sync_copy(zdeg, agg_sh.at[pl.ds(sid * RPT, RPT)])
    plsc.subcore_barrier()

    pltpu.sync_copy(st_hbm, s_tile)
    pltpu.sync_copy(src_hbm.at[sid], src_buf)
    pltpu.sync_copy(dst_hbm.at[sid], dst_buf)

    def sstart(j, b):
        pltpu.async_copy(vals_all.at[j], agg_sh.at[dst_buf.at[j]], sems[b],
                         add=True)

    def sfinish(j, b):
        pltpu.make_async_copy(vals_all.at[j], agg_sh.at[dst_buf.at[j]],
                              sems[b]).wait()

    def compute(j):
        for g in range(CH // L):
            idx = src_buf[j, pl.ds(g * L, L)]
            vals_all[j, pl.ds(g * L, L)] = plsc.load_gather(s_tile, [idx])

    def sbody(i, _):
        for k in range(NB):
            j = NB * i + k
            compute(j)

            @pl.when(j >= NB)
            def _():
                sfinish(j - NB, k)
            sstart(j, k)
        return 0
    lax.fori_loop(0, CPT // NB, sbody, 0)
    tail = (CPT // NB) * NB
    for j in range(tail, CPT):
        compute(j)
        sfinish(j - NB, j % NB)
        sstart(j, j % NB)
    for j in range(CPT - NB, CPT):
        sfinish(j, j % NB)
    plsc.subcore_barrier()

    pltpu.sync_copy(agg_sh.at[pl.ds(sid * RPT, RPT)], agg_t)
    pltpu.sync_copy(degc_hbm.at[pl.ds(sid * RPT, RPT)], deg_t)
    pltpu.sync_copy(s2rb_hbm.at[pl.ds(sid * RPT, RPT)], s2r_t)

    def ebody(k, _):
        a = agg_t[pl.ds(k * L, L)]
        d = deg_t[pl.ds(k * L, L)]
        out_t[pl.ds(k * L, L)] = a / d + s2r_t[pl.ds(k * L, L)]
        return 0
    lax.fori_loop(0, RPT // L, ebody, 0)
    pltpu.sync_copy(out_t, out_hbm.at[pl.ds(sid * RPT, RPT)])


def _tc_dense_body(acc_ref, deg0_ref, deg1_ref, x_ref, w1lt_ref, w1rt_ref,
                   b1l_ref, w2lt_ref, w2rt_ref, b2l_ref,
                   s2l_ref, s2r_ref, degc_ref):
    d = jnp.maximum(deg0_ref[...] + deg1_ref[...], 1.0)
    degc_ref[...] = d
    m0 = acc_ref[0] / d[:, None]
    m1 = acc_ref[1] / d[:, None]
    w1lt = w1lt_ref[...]
    h = (jnp.dot(m0, w1lt[:DH], preferred_element_type=jnp.float32)
         + jnp.dot(m1, w1lt[DH:], preferred_element_type=jnp.float32)
         + jnp.dot(x_ref[...], w1rt_ref[...], preferred_element_type=jnp.float32)
         + b1l_ref[...])
    h = jnp.maximum(h, 0.0)
    s2l_ref[...] = jnp.dot(h, w2lt_ref[...],
                           preferred_element_type=jnp.float32)[:, 0]
    s2r_ref[...] = (jnp.dot(h, w2rt_ref[...],
                            preferred_element_type=jnp.float32)[:, 0]
                    + b2l_ref[0, 0])


_tc_dense = pl.pallas_call(
    _tc_dense_body,
    grid=(NP // BN,),
    in_specs=[
        pl.BlockSpec((NC, BN, DH), lambda i: (0, i, 0)),
        pl.BlockSpec((BN,), lambda i: (i,)),
        pl.BlockSpec((BN,), lambda i: (i,)),
        pl.BlockSpec((BN, D), lambda i: (i, 0)),
        pl.BlockSpec((D, D), lambda i: (0, 0)),
        pl.BlockSpec((D, D), lambda i: (0, 0)),
        pl.BlockSpec((1, D), lambda i: (0, 0)),
        pl.BlockSpec((D, 1), lambda i: (0, 0)),
        pl.BlockSpec((D, 1), lambda i: (0, 0)),
        pl.BlockSpec((1, 1), lambda i: (0, 0)),
    ],
    out_specs=[
        pl.BlockSpec((BN,), lambda i: (i,)),
        pl.BlockSpec((BN,), lambda i: (i,)),
        pl.BlockSpec((BN,), lambda i: (i,)),
    ],
    out_shape=[
        jax.ShapeDtypeStruct((NP,), jnp.float32),
        jax.ShapeDtypeStruct((NP,), jnp.float32),
        jax.ShapeDtypeStruct((NP,), jnp.float32),
    ],
)


def kernel(x, edge_index, W1l, b1l, W1r, W2l, b2l, W2r):
    srcp = jnp.concatenate([edge_index[0],
                            jnp.zeros((EPAD,), jnp.int32)])
    dstp = jnp.concatenate([edge_index[1],
                            N + jnp.arange(EPAD, dtype=jnp.int32) % (NP - N)])
    srcA = (srcp * 2).reshape(NS, CPT, CH)
    srcB = (srcp * 2 + 1).reshape(NS, CPT, CH)
    dst4 = dstp.reshape(NS, CPT, CH)

    xr = x.reshape(2 * N, DH)
    acc, deg0, deg1 = _sc_agg_rows(xr, srcA, srcB, dst4)

    s2l, s2rb, degc = _tc_dense(acc, deg0, deg1, x,
                                W1l.T, W1r.T, b1l.reshape(1, D),
                                W2l.T, W2r.T, b2l.reshape(1, 1))

    st = jnp.stack([s2l, s2l], axis=1).reshape(2 * NP)
    out = _sc_agg_scalar(st, degc, s2rb, srcA, dst4)
    return out[:N]


# speedup vs baseline: 1.3127x; 1.3044x over previous
import functools

import jax
import jax.numpy as jnp
from jax import lax
from jax.experimental import pallas as pl
from jax.experimental.pallas import tpu as pltpu
from jax.experimental.pallas import tpu_sc as plsc

N = 10000
NP = 10240
D = 128
DH = D // 2
E = 320000
NC, NS, L = 2, 16, 16
NT = NC * NS
CH1 = 125
CPT1 = (E // NS) // CH1
CH2 = 80
CPT2 = (E // NS) // CH2
RPT = NP // NS
ZR = 128
BN = 1024
NB1 = 4
NB2 = 5

_mesh = plsc.VectorSubcoreMesh(core_axis_name="c", subcore_axis_name="s")
_mesh1 = plsc.VectorSubcoreMesh(core_axis_name="c", subcore_axis_name="s",
                                num_cores=1)


@functools.partial(
    pl.kernel,
    out_type=(
        jax.ShapeDtypeStruct((NC, NP, DH), jnp.float32),
        jax.ShapeDtypeStruct((NC, NP), jnp.float32),
    ),
    mesh=_mesh,
    scratch_types=[
        pltpu.VMEM((CPT1, CH1), jnp.int32),
        pltpu.VMEM((CPT1, CH1), jnp.int32),
    ] + [pltpu.VMEM((CH1, DH), jnp.float32)] * NB1 + [
        pltpu.VMEM((ZR,), jnp.float32),
        pltpu.VMEM((ZR, DH), jnp.float32),
        pltpu.VMEM((RPT,), jnp.float32),
        pltpu.VMEM_SHARED((NP, DH), jnp.float32),
        pltpu.VMEM_SHARED((NP,), jnp.float32),
    ] + [pltpu.SemaphoreType.DMA] * NB1,
    compiler_params=pltpu.CompilerParams(use_tc_tiling_on_sc=False),
)
def _sc_agg_rows(xr_hbm, srcA_hbm, srcB_hbm, dst_hbm, acc_out, deg_out,
                 src_buf, dst_buf, rows_0, rows_1, rows_2, rows_3,
                 ones_v, zrow, zdeg, acc_sh, deg_sh,
                 sem_0, sem_1, sem_2, sem_3):
    cid = lax.axis_index("c")
    sid = lax.axis_index("s")
    bufs = [rows_0, rows_1, rows_2, rows_3]
    sems = [sem_0, sem_1, sem_2, sem_3]

    def zfill(r, _):
        for k in range(DH // L):
            zrow[r, pl.ds(k * L, L)] = jnp.zeros((L,), jnp.float32)
        return 0
    lax.fori_loop(0, ZR, zfill, 0)
    for k in range(RPT // L):
        zdeg[pl.ds(k * L, L)] = jnp.zeros((L,), jnp.float32)
    for k in range(ZR // L):
        ones_v[pl.ds(k * L, L)] = jnp.ones((L,), jnp.float32)

    for k in range(RPT // ZR):
        pltpu.sync_copy(zrow, acc_sh.at[pl.ds(sid * RPT + k * ZR, ZR)])
    pltpu.sync_copy(zdeg, deg_sh.at[pl.ds(sid * RPT, RPT)])
    plsc.subcore_barrier()

    @pl.when(cid == 0)
    def _():
        pltpu.sync_copy(srcA_hbm.at[sid], src_buf)

    @pl.when(cid == 1)
    def _():
        pltpu.sync_copy(srcB_hbm.at[sid], src_buf)
    pltpu.sync_copy(dst_hbm.at[sid], dst_buf)

    def start(j, b):
        pltpu.async_copy(xr_hbm.at[src_buf.at[j]], bufs[b], sems[b])

    def finish(j, b):
        pltpu.make_async_copy(xr_hbm.at[src_buf.at[j]], bufs[b],
                              sems[b]).wait()

    for k in range(NB1 - 1):
        start(k, k)

    def body(i, _):
        for k in range(NB1):
            j = NB1 * i + k

            @pl.when(j + NB1 - 1 < CPT1)
            def _():
                start(j + NB1 - 1, (k + NB1 - 1) % NB1)
            finish(j, k)
            pltpu.sync_copy(bufs[k], acc_sh.at[dst_buf.at[j]], add=True)

            @pl.when(cid == (k % 2))
            def _():
                pltpu.sync_copy(ones_v.at[pl.ds(0, CH1)],
                                deg_sh.at[dst_buf.at[j]], add=True)
        return 0
    lax.fori_loop(0, CPT1 // NB1, body, 0)
    plsc.subcore_barrier()

    pltpu.sync_copy(acc_sh.at[pl.ds(sid * RPT, RPT)],
                    acc_out.at[cid, pl.ds(sid * RPT, RPT)])
    pltpu.sync_copy(deg_sh.at[pl.ds(sid * RPT, RPT)],
                    deg_out.at[cid, pl.ds(sid * RPT, RPT)])


@functools.partial(
    pl.kernel,
    out_type=jax.ShapeDtypeStruct((NP,), jnp.float32),
    mesh=_mesh1,
    scratch_types=[
        pltpu.VMEM((CPT2, CH2), jnp.int32),
        pltpu.VMEM((CPT2, CH2), jnp.int32),
        pltpu.VMEM((NP,), jnp.float32),
        pltpu.VMEM((CPT2, CH2), jnp.float32),
        pltpu.VMEM((RPT,), jnp.float32),
        pltpu.VMEM((RPT,), jnp.float32),
        pltpu.VMEM((RPT,), jnp.float32),
        pltpu.VMEM((RPT,), jnp.float32),
        pltpu.VMEM((RPT,), jnp.float32),
        pltpu.VMEM_SHARED((NP,), jnp.float32),
    ] + [pltpu.SemaphoreType.DMA] * NB2,
    compiler_params=pltpu.CompilerParams(use_tc_tiling_on_sc=False,
                                         needs_layout_passes=False),
)
def _sc_agg_scalar(s_hbm, degc_hbm, s2rb_hbm, src_hbm, dst_hbm, out_hbm,
                   src_buf, dst_buf, s_tile, vals_all, zdeg,
                   agg_t, deg_t, s2r_t, out_t, agg_sh,
                   sem_0, sem_1, sem_2, sem_3, sem_4):
    sid = lax.axis_index("s")
    sems = [sem_0, sem_1, sem_2, sem_3, sem_4]

    for k in range(RPT // L):
        zdeg[pl.ds(k * L, L)] = jnp.zeros((L,), jnp.float32)
    pltpu.sync_copy(zdeg, agg_sh.at[pl.ds(sid * RPT, RPT)])
    plsc.subcore_barrier()

    pltpu.sync_copy(s_hbm, s_tile)
    pltpu.sync_copy(src_hbm.at[sid], src_buf)
    pltpu.sync_copy(dst_hbm.at[sid], dst_buf)

    def sstart(j, b):
        pltpu.async_copy(vals_all.at[j], agg_sh.at[dst_buf.at[j]], sems[b],
                         add=True)

    def sfinish(j, b):
        pltpu.make_async_copy(vals_all.at[j], agg_sh.at[dst_buf.at[j]],
                              sems[b]).wait()

    def sbody(i, _):
        for k in range(NB2):
            j = NB2 * i + k
            for g in range(CH2 // L):
                idx = src_buf[j, pl.ds(g * L, L)]
                vals_all[j, pl.ds(g * L, L)] = plsc.load_gather(s_tile, [idx])

            @pl.when(j >= NB2)
            def _():
                sfinish(j - NB2, k)
            sstart(j, k)
        return 0
    lax.fori_loop(0, CPT2 // NB2, sbody, 0)
    for k in range(NB2):
        sfinish(CPT2 - NB2 + k, k)
    plsc.subcore_barrier()

    pltpu.sync_copy(agg_sh.at[pl.ds(sid * RPT, RPT)], agg_t)
    pltpu.sync_copy(degc_hbm.at[pl.ds(sid * RPT, RPT)], deg_t)
    pltpu.sync_copy(s2rb_hbm.at[pl.ds(sid * RPT, RPT)], s2r_t)

    def ebody(k, _):
        a = agg_t[pl.ds(k * L, L)]
        d = deg_t[pl.ds(k * L, L)]
        out_t[pl.ds(k * L, L)] = a / d + s2r_t[pl.ds(k * L, L)]
        return 0
    lax.fori_loop(0, RPT // L, ebody, 0)
    pltpu.sync_copy(out_t, out_hbm.at[pl.ds(sid * RPT, RPT)])


def _tc_dense_body(acc_ref, deg_ref, x_ref, w1lt_ref, w1rt_ref, b1l_ref,
                   w2lt_ref, w2rt_ref, b2l_ref, s2l_ref, s2r_ref, degc_ref):
    d = jnp.maximum(deg_ref[0] + deg_ref[1], 1.0)
    degc_ref[...] = d
    m0 = acc_ref[0] / d[:, None]
    m1 = acc_ref[1] / d[:, None]
    w1lt = w1lt_ref[...]
    h = (jnp.dot(m0, w1lt[:DH], preferred_element_type=jnp.float32)
         + jnp.dot(m1, w1lt[DH:], preferred_element_type=jnp.float32)
         + jnp.dot(x_ref[...], w1rt_ref[...], preferred_element_type=jnp.float32)
         + b1l_ref[...])
    h = jnp.maximum(h, 0.0)
    s2l_ref[...] = jnp.dot(h, w2lt_ref[...],
                           preferred_element_type=jnp.float32)[:, 0]
    s2r_ref[...] = (jnp.dot(h, w2rt_ref[...],
                            preferred_element_type=jnp.float32)[:, 0]
                    + b2l_ref[0, 0])


_tc_dense = pl.pallas_call(
    _tc_dense_body,
    grid=(NP // BN,),
    in_specs=[
        pl.BlockSpec((NC, BN, DH), lambda i: (0, i, 0)),
        pl.BlockSpec((NC, BN), lambda i: (0, i)),
        pl.BlockSpec((BN, D), lambda i: (i, 0)),
        pl.BlockSpec((D, D), lambda i: (0, 0)),
        pl.BlockSpec((D, D), lambda i: (0, 0)),
        pl.BlockSpec((1, D), lambda i: (0, 0)),
        pl.BlockSpec((D, 1), lambda i: (0, 0)),
        pl.BlockSpec((D, 1), lambda i: (0, 0)),
        pl.BlockSpec((1, 1), lambda i: (0, 0)),
    ],
    out_specs=[
        pl.BlockSpec((BN,), lambda i: (i,)),
        pl.BlockSpec((BN,), lambda i: (i,)),
        pl.BlockSpec((BN,), lambda i: (i,)),
    ],
    out_shape=[
        jax.ShapeDtypeStruct((NP,), jnp.float32),
        jax.ShapeDtypeStruct((NP,), jnp.float32),
        jax.ShapeDtypeStruct((NP,), jnp.float32),
    ],
)


def kernel(x, edge_index, W1l, b1l, W1r, W2l, b2l, W2r):
    src = edge_index[0]
    srcA = (src * 2).reshape(NS, CPT1, CH1)
    srcB = (src * 2 + 1).reshape(NS, CPT1, CH1)
    dst_a = edge_index[1].reshape(NS, CPT1, CH1)
    src_c = src.reshape(NS, CPT2, CH2)
    dst_c = edge_index[1].reshape(NS, CPT2, CH2)

    xr = x.reshape(2 * N, DH)
    acc, deg = _sc_agg_rows(xr, srcA, srcB, dst_a)

    s2l, s2rb, degc = _tc_dense(acc, deg, x,
                                W1l.T, W1r.T, b1l.reshape(1, D),
                                W2l.T, W2r.T, b2l.reshape(1, 1))

    out = _sc_agg_scalar(s2l, degc, s2rb, src_c, dst_c)
    return out[:N]
